# Initial kernel scaffold; baseline (speedup 1.0000x reference)
import jax
import jax.numpy as jnp
from jax.experimental import pallas as pl


def kernel(batch_pillar_features, batch_indices, sample_indices, batch_size):
    raise NotImplementedError("write your pallas kernel here")


# speedup vs baseline: 1.4497x; 1.4497x over previous
import functools

import jax
import jax.numpy as jnp
from jax import lax
from jax.experimental import pallas as pl
from jax.experimental.pallas import tpu as pltpu
from jax.experimental.pallas import tpu_sc as plsc

C = 64
NX = 432
NY = 496
B = 4
M = 48000
S = NX * NY
CHUNK = S // 8
MP = M + 8
SUB = 2976
NSUB = CHUNK // SUB
SUBV = SUB // 16
MV = M // 16

_BLK = 9600


def _prep_body(feats_ref, bidx_ref, sidx_ref, featsT_ref, dst_ref):
    f = feats_ref[...]
    featsT_ref[...] = f.T
    x = bidx_ref[:, 2]
    y = bidx_ref[:, 1]
    s = sidx_ref[:, 0]
    dst = s * S + (NX - 1 - x) * NY + y
    dst_ref[...] = dst[:, None]


def _prep(feats, bidx, sidx2d):
    return pl.pallas_call(
        _prep_body,
        grid=(M // _BLK,),
        in_specs=[
            pl.BlockSpec((_BLK, C), lambda i: (i, 0)),
            pl.BlockSpec((_BLK, 3), lambda i: (i, 0)),
            pl.BlockSpec((_BLK, 1), lambda i: (i, 0)),
        ],
        out_specs=[
            pl.BlockSpec((C, _BLK), lambda i: (0, i)),
            pl.BlockSpec((_BLK, 1), lambda i: (i, 0)),
        ],
        out_shape=[
            jax.ShapeDtypeStruct((C, M), jnp.float32),
            jax.ShapeDtypeStruct((M, 1), jnp.int32),
        ],
    )(feats, bidx, sidx2d)


@functools.partial(
    pl.kernel,
    mesh=plsc.VectorSubcoreMesh(core_axis_name="c", subcore_axis_name="s"),
    out_type=jax.ShapeDtypeStruct((B * C * S,), jnp.float32),
    compiler_params=pltpu.CompilerParams(needs_layout_passes=False),
    scratch_types=[
        pltpu.VMEM((CHUNK,), jnp.int32),
        pltpu.VMEM((SUB,), jnp.float32),
        pltpu.SemaphoreType.DMA,
        pltpu.SemaphoreType.DMA,
    ],
)
def _sc_scatter(featsT_hbm, dst_hbm, out_hbm, win_v, out_v, sem_f, sem_o):
    wid = lax.axis_index("s") * 2 + lax.axis_index("c")
    b = wid // 8
    j = wid % 8
    base = b * S + j * CHUNK

    fill = jnp.full((16,), M, jnp.int32)

    def init_body(i, _):
        win_v[pl.ds(i * 16, 16)] = fill
        return 0

    lax.fori_loop(0, CHUNK // 16, init_body, 0)

    def phase1(dst_v):
        pltpu.sync_copy(dst_hbm, dst_v)
        iota = lax.iota(jnp.int32, 16)

        def body(v, _):
            idx = dst_v[pl.ds(v * 16, 16)]
            m = iota + v * 16
            local = idx - base
            mask = (local >= 0) & (local < CHUNK)
            safe = jnp.where(mask, local, 0)
            plsc.store_scatter(win_v, [safe], m, mask=mask)
            return 0

        lax.fori_loop(0, MV, body, 0)

    pl.run_scoped(phase1, pltpu.VMEM((M,), jnp.int32))

    def phase2(feat_v):
        pltpu.make_async_copy(featsT_hbm.at[0], feat_v.at[0], sem_f).start()

        def chan(c, _):
            par = lax.rem(c, 2)
            pltpu.make_async_copy(featsT_hbm.at[c], feat_v.at[par], sem_f).wait()

            @pl.when(c < C - 1)
            def _():
                pltpu.make_async_copy(
                    featsT_hbm.at[c + 1], feat_v.at[1 - par], sem_f
                ).start()

            prow = jnp.zeros((16,), jnp.int32) + par
            out_base = (b * C + c) * S + j * CHUNK

            def sub_loop(sb, _):
                def vloop(v, _):
                    widx = win_v[pl.ds(sb * SUB + v * 16, 16)]
                    vals = plsc.load_gather(feat_v, [prow, widx])
                    out_v[pl.ds(v * 16, 16)] = vals
                    return 0

                lax.fori_loop(0, SUBV, vloop, 0)
                cp = pltpu.make_async_copy(
                    out_v, out_hbm.at[pl.ds(out_base + sb * SUB, SUB)], sem_o
                )
                cp.start()
                cp.wait()
                return 0

            lax.fori_loop(0, NSUB, sub_loop, 0)
            return 0

        lax.fori_loop(0, C, chan, 0)

    pl.run_scoped(phase2, pltpu.VMEM((2, MP), jnp.float32))


def kernel(batch_pillar_features, batch_indices, sample_indices, batch_size):
    del batch_size
    featsT, dst2d = _prep(
        batch_pillar_features,
        batch_indices.astype(jnp.int32),
        sample_indices.astype(jnp.int32).reshape(M, 1),
    )
    featsT_p = jnp.concatenate(
        [featsT, jnp.zeros((C, MP - M), jnp.float32)], axis=1
    )
    out_flat = _sc_scatter(featsT_p, dst2d.reshape(M))
    return out_flat.reshape(B, C, NX, NY)


# speedup vs baseline: 3.6450x; 2.5142x over previous
import functools

import jax
import jax.numpy as jnp
from jax import lax
from jax.experimental import pallas as pl
from jax.experimental.pallas import tpu as pltpu
from jax.experimental.pallas import tpu_sc as plsc

C = 64
NX = 432
NY = 496
B = 4
M = 48000
S = NX * NY
CHUNK = S // 8
MP = M + 8
SUB = 1488
NSUB = CHUNK // SUB
MV = M // 16

_BLK = 9600


def _prep_body(feats_ref, bidx_ref, sidx_ref, featsT_ref, dst_ref):
    f = feats_ref[...]
    featsT_ref[...] = f.T
    x = bidx_ref[:, 2]
    y = bidx_ref[:, 1]
    s = sidx_ref[:, 0]
    dst = s * S + (NX - 1 - x) * NY + y
    dst_ref[...] = dst[:, None]


def _prep(feats, bidx, sidx2d):
    return pl.pallas_call(
        _prep_body,
        grid=(M // _BLK,),
        in_specs=[
            pl.BlockSpec((_BLK, C), lambda i: (i, 0)),
            pl.BlockSpec((_BLK, 3), lambda i: (i, 0)),
            pl.BlockSpec((_BLK, 1), lambda i: (i, 0)),
        ],
        out_specs=[
            pl.BlockSpec((C, _BLK), lambda i: (0, i)),
            pl.BlockSpec((_BLK, 1), lambda i: (i, 0)),
        ],
        out_shape=[
            jax.ShapeDtypeStruct((C, M), jnp.float32),
            jax.ShapeDtypeStruct((M, 1), jnp.int32),
        ],
    )(feats, bidx, sidx2d)


@functools.partial(
    pl.kernel,
    mesh=plsc.VectorSubcoreMesh(core_axis_name="c", subcore_axis_name="s"),
    out_type=jax.ShapeDtypeStruct((B * C * S,), jnp.float32),
    compiler_params=pltpu.CompilerParams(needs_layout_passes=False),
    scratch_types=[
        pltpu.VMEM((CHUNK,), jnp.int32),
        pltpu.VMEM((2 * SUB,), jnp.float32),
        pltpu.SemaphoreType.DMA,
        pltpu.SemaphoreType.DMA,
        pltpu.SemaphoreType.DMA,
    ],
)
def _sc_scatter(featsT_hbm, dst_hbm, out_hbm, win_v, out_v, sem_f, sem_o0, sem_o1):
    wid = lax.axis_index("s") * 2 + lax.axis_index("c")
    b = wid // 8
    j = wid % 8
    base = b * S + j * CHUNK

    fill = jnp.full((16,), M, jnp.int32)

    def init_body(i, _):
        for k in range(6):
            win_v[pl.ds((i * 6 + k) * 16, 16)] = fill
        return 0

    lax.fori_loop(0, CHUNK // 96, init_body, 0)

    def phase1(dst_v):
        pltpu.sync_copy(dst_hbm, dst_v)
        iota = lax.iota(jnp.int32, 16)

        def body(v8, _):
            for k in range(8):
                v = v8 * 8 + k
                idx = dst_v[pl.ds(v * 16, 16)]
                m = iota + v * 16
                local = idx - base
                mask = (local >= 0) & (local < CHUNK)
                safe = jnp.where(mask, local, 0)
                plsc.store_scatter(win_v, [safe], m, mask=mask)
            return 0

        lax.fori_loop(0, MV // 8, body, 0)

    pl.run_scoped(phase1, pltpu.VMEM((M,), jnp.int32))

    def phase2(feat_v):
        pltpu.make_async_copy(featsT_hbm.at[0], feat_v.at[0], sem_f).start()

        def chan(c, _):
            par = lax.rem(c, 2)
            pltpu.make_async_copy(featsT_hbm.at[c], feat_v.at[par], sem_f).wait()

            @pl.when(c < C - 1)
            def _():
                pltpu.make_async_copy(
                    featsT_hbm.at[c + 1], feat_v.at[1 - par], sem_f
                ).start()

            prow = jnp.zeros((16,), jnp.int32) + par
            out_base = (b * C + c) * S + j * CHUNK

            for sb in range(NSUB):
                p = sb % 2
                sem_o = sem_o0 if p == 0 else sem_o1
                obase = p * SUB
                src = out_v.at[pl.ds(obase, SUB)]
                if sb >= 2:
                    pltpu.make_async_copy(
                        src, out_hbm.at[pl.ds(out_base + (sb - 2) * SUB, SUB)],
                        sem_o,
                    ).wait()
                else:

                    @pl.when(c > 0)
                    def _():
                        pltpu.make_async_copy(
                            src, out_hbm.at[pl.ds(out_base, SUB)], sem_o
                        ).wait()

                @plsc.parallel_loop(0, SUB, step=16, unroll=8)
                def _(i):
                    widx = win_v[pl.ds(sb * SUB + i, 16)]
                    vals = plsc.load_gather(feat_v, [prow, widx])
                    out_v[pl.ds(obase + i, 16)] = vals

                pltpu.make_async_copy(
                    src, out_hbm.at[pl.ds(out_base + sb * SUB, SUB)], sem_o
                ).start()
            return 0

        lax.fori_loop(0, C, chan, 0)

        for p, sem_o in ((0, sem_o0), (1, sem_o1)):
            pltpu.make_async_copy(
                out_v.at[pl.ds(p * SUB, SUB)],
                out_hbm.at[pl.ds(p * SUB, SUB)],
                sem_o,
            ).wait()

    pl.run_scoped(phase2, pltpu.VMEM((2, MP), jnp.float32))


def kernel(batch_pillar_features, batch_indices, sample_indices, batch_size):
    del batch_size
    featsT, dst2d = _prep(
        batch_pillar_features,
        batch_indices.astype(jnp.int32),
        sample_indices.astype(jnp.int32).reshape(M, 1),
    )
    featsT_p = jnp.concatenate(
        [featsT, jnp.zeros((C, MP - M), jnp.float32)], axis=1
    )
    out_flat = _sc_scatter(featsT_p, dst2d.reshape(M))
    return out_flat.reshape(B, C, NX, NY)
